# baseline (speedup 1.0000x reference)
import functools

import jax
import jax.numpy as jnp
from jax import lax
from jax.experimental import pallas as pl
from jax.experimental.pallas import tpu as pltpu
from jax.experimental.pallas import tpu_sc as plsc

N = 2048
V = 100000
D = 512
H = 512
BN = 64
NB = N // BN
NC, NS = 2, 16
NW = NC * NS
RW = N // NW
LN = 16
DC = D // LN


def _sweep_body(pred_ref, tgt_ref, max_ref, pos_ref, nll_ref):
    x = pred_ref[...]
    tgt = tgt_ref[0, 0, :]
    col = lax.broadcasted_iota(jnp.int32, (BN, V), 1)
    m = jnp.max(x, axis=1)
    pos = jnp.min(jnp.where(x == m[:, None], col, V), axis=1)
    nll = -jnp.sum(jnp.where(col == tgt[:, None], x, 0.0), axis=1)
    max_ref[0, 0, :] = m
    pos_ref[0, 0, :] = pos
    nll_ref[0, 0, :] = nll


def _sweep(pred_ll, target3):
    return pl.pallas_call(
        _sweep_body,
        grid=(NB,),
        in_specs=[
            pl.BlockSpec((BN, V), lambda i: (i, 0)),
            pl.BlockSpec((1, 1, BN), lambda i: (i, 0, 0)),
        ],
        out_specs=[
            pl.BlockSpec((1, 1, BN), lambda i: (i, 0, 0)),
            pl.BlockSpec((1, 1, BN), lambda i: (i, 0, 0)),
            pl.BlockSpec((1, 1, BN), lambda i: (i, 0, 0)),
        ],
        out_shape=[
            jax.ShapeDtypeStruct((NB, 1, BN), jnp.float32),
            jax.ShapeDtypeStruct((NB, 1, BN), jnp.int32),
            jax.ShapeDtypeStruct((NB, 1, BN), jnp.float32),
        ],
        compiler_params=pltpu.CompilerParams(
            dimension_semantics=("arbitrary",),
        ),
    )(pred_ll, target3)


def _sc_distance(target, pos, emb_table):
    mesh = plsc.VectorSubcoreMesh(
        core_axis_name="c", subcore_axis_name="s", num_cores=NC, num_subcores=NS
    )

    @functools.partial(
        pl.kernel,
        out_type=jax.ShapeDtypeStruct((N, LN), jnp.float32),
        mesh=mesh,
        scratch_types=[
            pltpu.VMEM((RW,), jnp.int32),
            pltpu.VMEM((RW,), jnp.int32),
            pltpu.VMEM((RW, D), jnp.float32),
            pltpu.VMEM((RW, D), jnp.float32),
            pltpu.VMEM((RW, LN), jnp.float32),
            pltpu.SemaphoreType.DMA,
            pltpu.SemaphoreType.DMA,
        ],
    )
    def sc_k(tgt_hbm, pos_hbm, emb_hbm, out_hbm,
             tgt_v, pos_v, gold_v, pred_v, d2_v, sem1, sem2):
        wid = lax.axis_index("s") * NC + lax.axis_index("c")
        base = wid * RW
        pltpu.sync_copy(tgt_hbm.at[pl.ds(base, RW)], tgt_v)
        pltpu.sync_copy(pos_hbm.at[pl.ds(base, RW)], pos_v)
        cp1 = pltpu.async_copy(emb_hbm.at[tgt_v], gold_v, sem1)
        cp2 = pltpu.async_copy(emb_hbm.at[pos_v], pred_v, sem2)
        cp1.wait()
        cp2.wait()

        def row(r, carry):
            acc = jnp.zeros((LN,), jnp.float32)
            for c in range(DC):
                g = gold_v[r, pl.ds(c * LN, LN)]
                p = pred_v[r, pl.ds(c * LN, LN)]
                dlt = g - p + 1e-6
                acc = acc + dlt * dlt
            d2_v[r, :] = acc
            return carry

        lax.fori_loop(0, RW, row, 0)
        pltpu.sync_copy(d2_v, out_hbm.at[pl.ds(base, RW)])

    return sc_k(target, pos, emb_table)


def _epilogue_body(d2_ref, nll_ref, pmax_ref, tgt_ref, w1w_ref, w1b_ref,
                   w2w_ref, w2b_ref, loss_ref, nlls_ref):
    dist = jnp.sqrt(jnp.sum(d2_ref[...], axis=1, keepdims=True))
    h = jnp.maximum(dist * w1w_ref[...] + w1b_ref[...], 0.0)
    md = jnp.sum(h * w2w_ref[...], axis=1, keepdims=True) + w2b_ref[0, 0]
    x = jax.nn.sigmoid(md) * 0.5
    mask = (tgt_ref[...] != 0).astype(jnp.float32)
    nll_m = nll_ref[...] * mask
    pred_m = -pmax_ref[...] * mask
    loss = (0.5 + x) * nll_m + (0.5 - x) * pred_m
    loss_ref[0, 0] = jnp.sum(loss)
    nlls_ref[0, 0] = jnp.sum(nll_m)


def _epilogue(dist2, nll, pmax, target, w1_W, w1_b, w2_W, w2_b):
    return pl.pallas_call(
        _epilogue_body,
        in_specs=[
            pl.BlockSpec((N, LN), lambda: (0, 0)),
            pl.BlockSpec((N, 1), lambda: (0, 0)),
            pl.BlockSpec((N, 1), lambda: (0, 0)),
            pl.BlockSpec((N, 1), lambda: (0, 0)),
            pl.BlockSpec((1, H), lambda: (0, 0)),
            pl.BlockSpec((1, H), lambda: (0, 0)),
            pl.BlockSpec((1, H), lambda: (0, 0)),
            pl.BlockSpec((1, 1), lambda: (0, 0)),
        ],
        out_specs=[
            pl.BlockSpec(memory_space=pltpu.SMEM),
            pl.BlockSpec(memory_space=pltpu.SMEM),
        ],
        out_shape=[
            jax.ShapeDtypeStruct((1, 1), jnp.float32),
            jax.ShapeDtypeStruct((1, 1), jnp.float32),
        ],
    )(dist2, nll[:, None], pmax[:, None], target[:, None],
      w1_W.reshape(1, H), w1_b.reshape(1, H), w2_W.reshape(1, H),
      w2_b.reshape(1, 1))


def kernel(pred_ll, target, emb_table, w1_W, w1_b, w2_W, w2_b):
    target3 = target.reshape(NB, 1, BN)
    m3, p3, n3 = _sweep(pred_ll, target3)
    pmax = m3.reshape(N)
    pos = p3.reshape(N)
    nll = n3.reshape(N)
    dist2 = _sc_distance(target, pos, emb_table)
    loss, nll_sum = _epilogue(dist2, nll, pmax, target, w1_W, w1_b, w2_W, w2_b)
    return (loss[0, 0], nll_sum[0, 0])
